# baseline (speedup 1.0000x reference)
import functools

import jax
import jax.numpy as jnp
from jax import lax
from jax.experimental import pallas as pl
from jax.experimental.pallas import tpu as pltpu
from jax.experimental.pallas import tpu_sc as plsc

_LAM = 0.9
_B = 16
_H = 512
_W = 512
_N = _H * _W
_L = 16
_NB0 = 2048
_NB12 = 1024
_ST0 = _NB0 + 1
_ST12 = _NB12 + 1
_HISTW = _L * _NB0 + 128
_ROWS = 32
_CHP = _ROWS * _W
_NCH = (_N // 2) // _CHP
_UN = 8


def _iota():
    return lax.iota(jnp.int32, _L)


def _merge_hist(hist_ref, hc_ref, nb, stride):
    def body(j, c):
        acc = hist_ref[pl.ds(j * _L, _L)]
        for r in range(1, _L):
            acc = acc + hist_ref[pl.ds(r * stride + j * _L, _L)]
        hc_ref[pl.ds(j * _L, _L)] = acc
        return c

    lax.fori_loop(0, nb // _L, body, 0)


def _total(hc_ref, nb):
    def body(j, acc):
        return acc + jnp.sum(hc_ref[pl.ds(j * _L, _L)])

    return lax.fori_loop(0, nb // _L, body, jnp.float32(0.0))


def _scan_level(hc_ref, nb, target, sum_above):
    nv = nb // _L
    iota = _iota()

    def body(i, carry):
        best, acc = carry
        ii = nv - 1 - i
        s = hc_ref[pl.ds(ii * _L, _L)]
        p = plsc.cumsum(s)
        tot = jnp.sum(s)
        csum = sum_above + acc + (tot - p + s)
        gbin = ii * _L + iota
        cand = jnp.max(jnp.where(csum > target, gbin, jnp.int32(-1)))
        return jnp.maximum(best, cand), acc + tot

    best, _ = lax.fori_loop(0, nv, body, (jnp.int32(-1), jnp.float32(0.0)))
    tstar = jnp.maximum(best, 0)

    def body2(i, acc):
        s = hc_ref[pl.ds(i * _L, _L)]
        gbin = i * _L + iota
        return acc + jnp.sum(jnp.where(gbin > tstar, s, jnp.float32(0.0)))

    above = lax.fori_loop(0, nv, body2, jnp.float32(0.0))
    return best, tstar, sum_above + above


def _merge_partner(hc_ref, pbuf_ref, sh_ref, sid, nb):
    pltpu.sync_copy(hc_ref.at[pl.ds(0, nb)], sh_ref.at[sid, pl.ds(0, nb)])
    plsc.subcore_barrier()
    pltpu.sync_copy(sh_ref.at[sid ^ 1, pl.ds(0, nb)],
                    pbuf_ref.at[pl.ds(0, nb)])
    plsc.subcore_barrier()

    def body(j, c):
        hc_ref[pl.ds(j * _L, _L)] = (hc_ref[pl.ds(j * _L, _L)] +
                                     pbuf_ref[pl.ds(j * _L, _L)])
        return c

    lax.fori_loop(0, nb // _L, body, 0)


def _sc_body(power_hbm, thr_hbm, in_a, in_b, hist_ref, hc_ref, pbuf_ref,
             out_ref, sh_ref, sem_a, sem_b):
    c_id = lax.axis_index("c")
    sid = lax.axis_index("s")
    b = c_id * 8 + lax.shift_right_logical(sid, 1)
    half = sid & 1
    iota = _iota()
    lane0 = iota * _ST0
    lane12 = iota * _ST12
    base_r = half * (_H // 2)

    def psrc(c):
        return power_hbm.at[b, pl.ds(base_r + c * _ROWS, _ROWS), :]

    def zero_hist(nwords):
        z = jnp.zeros((_L,), jnp.float32)

        def zb(j, c):
            for u in range(8):
                hist_ref[pl.ds((j * 8 + u) * _L, _L)] = z
            return c

        lax.fori_loop(0, nwords // (_L * 8), zb, 0)

    def stream_pass(shift_hi, path_hi, shift_lo, nb, lane_base):
        def process(in_ref):
            def vb(j, c):
                rcs = []
                for u in range(_UN):
                    jj = j * _UN + u
                    rcs.append((lax.shift_right_logical(jj, 5),
                                (jj & 31) * _L))
                ws = [in_ref[r, pl.ds(c0, _L)] for r, c0 in rcs]
                uis = [plsc.bitcast(w, jnp.int32) for w in ws]
                ms = [lax.shift_right_logical(ui, shift_hi) == path_hi
                      for ui in uis]
                ixs = [lane_base + (lax.shift_right_logical(ui, shift_lo)
                                    & (nb - 1)) for ui in uis]
                for w, m, ix in zip(ws, ms, ixs):
                    plsc.addupdate_scatter(hist_ref, [ix], w, mask=m)
                return c

            lax.fori_loop(0, _CHP // _L // _UN, vb, 0)

        pltpu.async_copy(psrc(0), in_a, sem_a)

        def bodyr(g, carry):
            pltpu.async_copy(psrc(2 * g + 1), in_b, sem_b)
            pltpu.make_async_copy(psrc(2 * g), in_a, sem_a).wait()
            process(in_a)

            @pl.when(g + 1 < _NCH // 2)
            def _():
                pltpu.async_copy(psrc(2 * g + 2), in_a, sem_a)

            pltpu.make_async_copy(psrc(2 * g + 1), in_b, sem_b).wait()
            process(in_b)
            return carry

        lax.fori_loop(0, _NCH // 2, bodyr, 0)

    zero_hist(_HISTW)
    stream_pass(31, jnp.int32(0), 20, _NB0, lane0)
    _merge_hist(hist_ref, hc_ref, _NB0, _ST0)
    _merge_partner(hc_ref, pbuf_ref, sh_ref, sid, _NB0)
    total = _total(hc_ref, _NB0)
    target = jnp.float32(_LAM) * (total + jnp.float32(1e-10))
    best0, t0, sa = _scan_level(hc_ref, _NB0, target, jnp.float32(0.0))

    def refine(shift_hi, path_hi, shift_lo, sum_above):
        zero_hist(_L * _NB12 + 128)
        stream_pass(shift_hi, path_hi, shift_lo, _NB12, lane12)
        _merge_hist(hist_ref, hc_ref, _NB12, _ST12)
        _merge_partner(hc_ref, pbuf_ref, sh_ref, sid, _NB12)
        _, t, sa2 = _scan_level(hc_ref, _NB12, target, sum_above)
        return t, sa2

    t1, sa = refine(20, t0, 10, sa)
    path01 = (t0 << 10) | t1
    t2, sa = refine(10, path01, 0, sa)

    vbits = jnp.full((_L,), (path01 << 10) | t2, jnp.int32)
    vf = plsc.bitcast(vbits, jnp.float32)
    include = (jnp.full((_L,), target - sa) >= vf) | jnp.full(
        (_L,), sa <= jnp.float32(0.0))
    thr_bits = vbits + jnp.where(include, jnp.int32(0), jnp.int32(1))
    thrf = plsc.bitcast(thr_bits, jnp.float32)
    no_cross = jnp.full((_L,), best0 < jnp.int32(0))
    thrf = jnp.where(no_cross, jnp.zeros((_L,), jnp.float32), thrf)
    out_ref[...] = thrf

    @pl.when(half == 0)
    def _():
        pltpu.sync_copy(out_ref, thr_hbm.at[b])


def _sc_select(power):
    mesh = plsc.VectorSubcoreMesh(core_axis_name="c", subcore_axis_name="s")
    f = functools.partial(
        pl.kernel,
        out_type=jax.ShapeDtypeStruct((_B, _L), jnp.float32),
        mesh=mesh,
        compiler_params=pltpu.CompilerParams(needs_layout_passes=False),
        scratch_types=[
            pltpu.VMEM((_ROWS, _W), jnp.float32),
            pltpu.VMEM((_ROWS, _W), jnp.float32),
            pltpu.VMEM((_HISTW,), jnp.float32),
            pltpu.VMEM((_NB0,), jnp.float32),
            pltpu.VMEM((_NB0,), jnp.float32),
            pltpu.VMEM((_L,), jnp.float32),
            pltpu.VMEM_SHARED((_L, _NB0), jnp.float32),
            pltpu.SemaphoreType.DMA,
            pltpu.SemaphoreType.DMA,
        ],
    )(_sc_body)
    return f(power)


def _power_body(xp_ref, o_ref):
    xx = xp_ref[...]
    e = xx[:, :, 0, :]
    o = xx[:, :, 1, :]
    o_ref[...] = e * e + o * o


def _power_call(x):
    xp = jnp.transpose(x, (0, 1, 3, 2))
    return pl.pallas_call(
        _power_body,
        grid=(_B, 8),
        in_specs=[pl.BlockSpec((1, _H // 8, 2, _W),
                               lambda b, i: (b, i, 0, 0))],
        out_specs=pl.BlockSpec((1, _H // 8, _W), lambda b, i: (b, i, 0)),
        out_shape=jax.ShapeDtypeStruct((_B, _H, _W), jnp.float32),
    )(xp)


def _mask_body(thr_ref, p_ref, o_ref):
    b = pl.program_id(0)
    t = thr_ref[b, 0]
    o_ref[...] = (p_ref[...] >= t).astype(jnp.float32)


def _mask_call(thr, power):
    return pl.pallas_call(
        _mask_body,
        grid=(_B,),
        in_specs=[
            pl.BlockSpec(memory_space=pltpu.SMEM),
            pl.BlockSpec((1, _H, _W), lambda b: (b, 0, 0)),
        ],
        out_specs=pl.BlockSpec((1, _H, _W), lambda b: (b, 0, 0)),
        out_shape=jax.ShapeDtypeStruct((_B, _H, _W), jnp.float32),
    )(thr, power)


def kernel(x):
    b, h, w, _ = x.shape
    power = _power_call(x)
    thr = _sc_select(power)
    maskf = _mask_call(thr, power)
    return maskf.reshape(b, h, w, 1)


# speedup vs baseline: 1.5265x; 1.5265x over previous
import functools

import jax
import jax.numpy as jnp
from jax import lax
from jax.experimental import pallas as pl
from jax.experimental.pallas import tpu as pltpu
from jax.experimental.pallas import tpu_sc as plsc

_LAM = 0.9
_B = 16
_H = 512
_W = 512
_N = _H * _W
_L = 16
_NB0 = 2048
_NB12 = 1024
_ST0 = _NB0 + 1
_ST12 = _NB12 + 1
_HISTW = _L * _NB0 + 128
_ROWS = 32
_CHP = _ROWS * _W
_NCH = (_N // 2) // _CHP
_XR = 16
_NCH0 = 256 // _XR
_UN = 8


def _iota():
    return lax.iota(jnp.int32, _L)


def _merge_hist(hist_ref, hc_ref, nb, stride):
    def body(j, c):
        acc = hist_ref[pl.ds(j * _L, _L)]
        for r in range(1, _L):
            acc = acc + hist_ref[pl.ds(r * stride + j * _L, _L)]
        hc_ref[pl.ds(j * _L, _L)] = acc
        return c

    lax.fori_loop(0, nb // _L, body, 0)


def _total(hc_ref, nb):
    def body(j, acc):
        return acc + jnp.sum(hc_ref[pl.ds(j * _L, _L)])

    return lax.fori_loop(0, nb // _L, body, jnp.float32(0.0))


def _scan_level(hc_ref, nb, target, sum_above):
    nv = nb // _L
    iota = _iota()

    def body(i, carry):
        best, acc = carry
        ii = nv - 1 - i
        s = hc_ref[pl.ds(ii * _L, _L)]
        p = plsc.cumsum(s)
        tot = jnp.sum(s)
        csum = sum_above + acc + (tot - p + s)
        gbin = ii * _L + iota
        cand = jnp.max(jnp.where(csum > target, gbin, jnp.int32(-1)))
        return jnp.maximum(best, cand), acc + tot

    best, _ = lax.fori_loop(0, nv, body, (jnp.int32(-1), jnp.float32(0.0)))
    tstar = jnp.maximum(best, 0)

    def body2(i, acc):
        s = hc_ref[pl.ds(i * _L, _L)]
        gbin = i * _L + iota
        return acc + jnp.sum(jnp.where(gbin > tstar, s, jnp.float32(0.0)))

    above = lax.fori_loop(0, nv, body2, jnp.float32(0.0))
    return best, tstar, sum_above + above


def _merge_partner(hc_ref, pbuf_ref, sh_ref, sid, nb):
    pltpu.sync_copy(hc_ref.at[pl.ds(0, nb)], sh_ref.at[sid, pl.ds(0, nb)])
    plsc.subcore_barrier()
    pltpu.sync_copy(sh_ref.at[sid ^ 1, pl.ds(0, nb)],
                    pbuf_ref.at[pl.ds(0, nb)])
    plsc.subcore_barrier()

    def body(j, c):
        hc_ref[pl.ds(j * _L, _L)] = (hc_ref[pl.ds(j * _L, _L)] +
                                     pbuf_ref[pl.ds(j * _L, _L)])
        return c

    lax.fori_loop(0, nb // _L, body, 0)


def _sc_body(xp_hbm, power_hbm, thr_hbm, in_a, in_b, xa, xb, pwa, pwb,
             hist_ref, hc_ref, pbuf_ref, out_ref, sh_ref, sem_a, sem_b,
             sem_xa, sem_xb, sem_wa, sem_wb):
    c_id = lax.axis_index("c")
    sid = lax.axis_index("s")
    b = c_id * 8 + lax.shift_right_logical(sid, 1)
    half = sid & 1
    iota = _iota()
    lane0 = iota * _ST0
    lane12 = iota * _ST12
    base_r = half * (_H // 2)

    def psrc(c):
        return power_hbm.at[b, pl.ds(base_r + c * _ROWS, _ROWS), :]

    def xsrc(c):
        return xp_hbm.at[b, pl.ds(base_r + c * _XR, _XR), :, :]

    def pdst(c):
        return power_hbm.at[b, pl.ds(base_r + c * _XR, _XR), :]

    def zero_hist(nwords):
        z = jnp.zeros((_L,), jnp.float32)

        def zb(j, c):
            for u in range(8):
                hist_ref[pl.ds((j * 8 + u) * _L, _L)] = z
            return c

        lax.fori_loop(0, nwords // (_L * 8), zb, 0)

    def stream_pass(shift_hi, path_hi, shift_lo, nb, lane_base):
        def process(in_ref):
            def vb(j, c):
                rcs = []
                for u in range(_UN):
                    jj = j * _UN + u
                    rcs.append((lax.shift_right_logical(jj, 5),
                                (jj & 31) * _L))
                ws = [in_ref[r, pl.ds(c0, _L)] for r, c0 in rcs]
                uis = [plsc.bitcast(w, jnp.int32) for w in ws]
                ms = [lax.shift_right_logical(ui, shift_hi) == path_hi
                      for ui in uis]
                ixs = [lane_base + (lax.shift_right_logical(ui, shift_lo)
                                    & (nb - 1)) for ui in uis]
                for w, m, ix in zip(ws, ms, ixs):
                    plsc.addupdate_scatter(hist_ref, [ix], w, mask=m)
                return c

            lax.fori_loop(0, _CHP // _L // _UN, vb, 0)

        pltpu.async_copy(psrc(0), in_a, sem_a)

        def bodyr(g, carry):
            pltpu.async_copy(psrc(2 * g + 1), in_b, sem_b)
            pltpu.make_async_copy(psrc(2 * g), in_a, sem_a).wait()
            process(in_a)

            @pl.when(g + 1 < _NCH // 2)
            def _():
                pltpu.async_copy(psrc(2 * g + 2), in_a, sem_a)

            pltpu.make_async_copy(psrc(2 * g + 1), in_b, sem_b).wait()
            process(in_b)
            return carry

        lax.fori_loop(0, _NCH // 2, bodyr, 0)

    def process0(in_ref, pw_ref):
        def vb(j, c):
            rcs = []
            for u in range(_UN):
                jj = j * _UN + u
                rcs.append((lax.shift_right_logical(jj, 5),
                            (jj & 31) * _L))
            es = [in_ref[r, 0, pl.ds(c0, _L)] for r, c0 in rcs]
            os_ = [in_ref[r, 1, pl.ds(c0, _L)] for r, c0 in rcs]
            ws = [e * e + o * o for e, o in zip(es, os_)]
            ixs = [lane0 + lax.shift_right_logical(
                plsc.bitcast(w, jnp.int32), 20) for w in ws]
            for (r, c0), w in zip(rcs, ws):
                pw_ref[r, pl.ds(c0, _L)] = w
            for w, ix in zip(ws, ixs):
                plsc.addupdate_scatter(hist_ref, [ix], w)
            return c

        lax.fori_loop(0, (_XR * _W) // _L // _UN, vb, 0)

    zero_hist(_HISTW)
    pltpu.async_copy(xsrc(0), xa, sem_xa)

    def body0(g, carry):
        pltpu.async_copy(xsrc(2 * g + 1), xb, sem_xb)
        pltpu.make_async_copy(xsrc(2 * g), xa, sem_xa).wait()

        @pl.when(g > 0)
        def _():
            pltpu.make_async_copy(pwa, pdst(2 * g - 2), sem_wa).wait()

        process0(xa, pwa)
        pltpu.async_copy(pwa, pdst(2 * g), sem_wa)

        @pl.when(g + 1 < _NCH0 // 2)
        def _():
            pltpu.async_copy(xsrc(2 * g + 2), xa, sem_xa)

        pltpu.make_async_copy(xsrc(2 * g + 1), xb, sem_xb).wait()

        @pl.when(g > 0)
        def _():
            pltpu.make_async_copy(pwb, pdst(2 * g - 1), sem_wb).wait()

        process0(xb, pwb)
        pltpu.async_copy(pwb, pdst(2 * g + 1), sem_wb)
        return carry

    lax.fori_loop(0, _NCH0 // 2, body0, 0)
    pltpu.make_async_copy(pwa, pdst(_NCH0 - 2), sem_wa).wait()
    pltpu.make_async_copy(pwb, pdst(_NCH0 - 1), sem_wb).wait()
    _merge_hist(hist_ref, hc_ref, _NB0, _ST0)
    _merge_partner(hc_ref, pbuf_ref, sh_ref, sid, _NB0)
    total = _total(hc_ref, _NB0)
    target = jnp.float32(_LAM) * (total + jnp.float32(1e-10))
    best0, t0, sa = _scan_level(hc_ref, _NB0, target, jnp.float32(0.0))

    def refine(shift_hi, path_hi, shift_lo, sum_above):
        zero_hist(_L * _NB12 + 128)
        stream_pass(shift_hi, path_hi, shift_lo, _NB12, lane12)
        _merge_hist(hist_ref, hc_ref, _NB12, _ST12)
        _merge_partner(hc_ref, pbuf_ref, sh_ref, sid, _NB12)
        _, t, sa2 = _scan_level(hc_ref, _NB12, target, sum_above)
        return t, sa2

    t1, sa = refine(20, t0, 10, sa)
    path01 = (t0 << 10) | t1
    t2, sa = refine(10, path01, 0, sa)

    vbits = jnp.full((_L,), (path01 << 10) | t2, jnp.int32)
    vf = plsc.bitcast(vbits, jnp.float32)
    include = (jnp.full((_L,), target - sa) >= vf) | jnp.full(
        (_L,), sa <= jnp.float32(0.0))
    thr_bits = vbits + jnp.where(include, jnp.int32(0), jnp.int32(1))
    thrf = plsc.bitcast(thr_bits, jnp.float32)
    no_cross = jnp.full((_L,), best0 < jnp.int32(0))
    thrf = jnp.where(no_cross, jnp.zeros((_L,), jnp.float32), thrf)
    out_ref[...] = thrf

    @pl.when(half == 0)
    def _():
        pltpu.sync_copy(out_ref, thr_hbm.at[b])


def _sc_select(xp):
    mesh = plsc.VectorSubcoreMesh(core_axis_name="c", subcore_axis_name="s")
    f = functools.partial(
        pl.kernel,
        out_type=(
            jax.ShapeDtypeStruct((_B, _H, _W), jnp.float32),
            jax.ShapeDtypeStruct((_B, _L), jnp.float32),
        ),
        mesh=mesh,
        compiler_params=pltpu.CompilerParams(needs_layout_passes=False),
        scratch_types=[
            pltpu.VMEM((_ROWS, _W), jnp.float32),
            pltpu.VMEM((_ROWS, _W), jnp.float32),
            pltpu.VMEM((_XR, 2, _W), jnp.float32),
            pltpu.VMEM((_XR, 2, _W), jnp.float32),
            pltpu.VMEM((_XR, _W), jnp.float32),
            pltpu.VMEM((_XR, _W), jnp.float32),
            pltpu.VMEM((_HISTW,), jnp.float32),
            pltpu.VMEM((_NB0,), jnp.float32),
            pltpu.VMEM((_NB0,), jnp.float32),
            pltpu.VMEM((_L,), jnp.float32),
            pltpu.VMEM_SHARED((_L, _NB0), jnp.float32),
        ] + [pltpu.SemaphoreType.DMA] * 6,
    )(_sc_body)
    return f(xp)


def _mask_body(thr_ref, p_ref, o_ref):
    b = pl.program_id(0)
    t = thr_ref[b, 0]
    o_ref[...] = (p_ref[...] >= t).astype(jnp.float32)


def _mask_call(thr, power):
    return pl.pallas_call(
        _mask_body,
        grid=(_B,),
        in_specs=[
            pl.BlockSpec(memory_space=pltpu.SMEM),
            pl.BlockSpec((1, _H, _W), lambda b: (b, 0, 0)),
        ],
        out_specs=pl.BlockSpec((1, _H, _W), lambda b: (b, 0, 0)),
        out_shape=jax.ShapeDtypeStruct((_B, _H, _W), jnp.float32),
    )(thr, power)


def kernel(x):
    b, h, w, _ = x.shape
    xp = jnp.transpose(x, (0, 1, 3, 2))
    power, thr = _sc_select(xp)
    maskf = _mask_call(thr, power)
    return maskf.reshape(b, h, w, 1)


# speedup vs baseline: 1.5356x; 1.0059x over previous
import functools

import jax
import jax.numpy as jnp
from jax import lax
from jax.experimental import pallas as pl
from jax.experimental.pallas import tpu as pltpu
from jax.experimental.pallas import tpu_sc as plsc

_LAM = 0.9
_B = 16
_H = 512
_W = 512
_N = _H * _W
_L = 16
_NB0 = 2048
_NB12 = 1024
_ST0 = _NB0 + 1
_ST12 = _NB12 + 1
_HISTW = _L * _NB0 + 128
_ROWS = 32
_CHP = _ROWS * _W
_NCH = (_N // 2) // _CHP
_XR = 16
_NCH0 = 256 // _XR
_UN = 8


def _iota():
    return lax.iota(jnp.int32, _L)


def _merge_hist(hist_ref, hc_ref, nb, stride):
    def body(j, c):
        acc = hist_ref[pl.ds(j * _L, _L)]
        for r in range(1, _L):
            acc = acc + hist_ref[pl.ds(r * stride + j * _L, _L)]
        hc_ref[pl.ds(j * _L, _L)] = acc
        return c

    lax.fori_loop(0, nb // _L, body, 0)


def _total(hc_ref, nb):
    def body(j, acc):
        return acc + jnp.sum(hc_ref[pl.ds(j * _L, _L)])

    return lax.fori_loop(0, nb // _L, body, jnp.float32(0.0))


def _scan_level(hc_ref, nb, target, sum_above):
    nv = nb // _L
    iota = _iota()

    def body(i, carry):
        best, acc = carry
        ii = nv - 1 - i
        s = hc_ref[pl.ds(ii * _L, _L)]
        p = plsc.cumsum(s)
        tot = jnp.sum(s)
        csum = sum_above + acc + (tot - p + s)
        gbin = ii * _L + iota
        cand = jnp.max(jnp.where(csum > target, gbin, jnp.int32(-1)))
        return jnp.maximum(best, cand), acc + tot

    best, _ = lax.fori_loop(0, nv, body, (jnp.int32(-1), jnp.float32(0.0)))
    tstar = jnp.maximum(best, 0)

    def body2(i, acc):
        s = hc_ref[pl.ds(i * _L, _L)]
        gbin = i * _L + iota
        return acc + jnp.sum(jnp.where(gbin > tstar, s, jnp.float32(0.0)))

    above = lax.fori_loop(0, nv, body2, jnp.float32(0.0))
    return best, tstar, sum_above + above


def _merge_partner(hc_ref, pbuf_ref, sh_ref, sid, nb):
    pltpu.sync_copy(hc_ref.at[pl.ds(0, nb)], sh_ref.at[sid, pl.ds(0, nb)])
    plsc.subcore_barrier()
    pltpu.sync_copy(sh_ref.at[sid ^ 1, pl.ds(0, nb)],
                    pbuf_ref.at[pl.ds(0, nb)])
    plsc.subcore_barrier()

    def body(j, c):
        hc_ref[pl.ds(j * _L, _L)] = (hc_ref[pl.ds(j * _L, _L)] +
                                     pbuf_ref[pl.ds(j * _L, _L)])
        return c

    lax.fori_loop(0, nb // _L, body, 0)


def _sc_body(xp_hbm, power_hbm, mask_hbm, in_a, in_b, xa, xb, pwa, pwb,
             hist_ref, hc_ref, pbuf_ref, out_ref, sh_ref, sem_a, sem_b,
             sem_xa, sem_xb, sem_wa, sem_wb):
    c_id = lax.axis_index("c")
    sid = lax.axis_index("s")
    b = c_id * 8 + lax.shift_right_logical(sid, 1)
    half = sid & 1
    iota = _iota()
    lane0 = iota * _ST0
    lane12 = iota * _ST12
    base_r = half * (_H // 2)

    def psrc(c):
        return power_hbm.at[b, pl.ds(base_r + c * _ROWS, _ROWS), :]

    def xsrc(c):
        return xp_hbm.at[b, pl.ds(base_r + c * _XR, _XR), :, :]

    def pdst(c):
        return power_hbm.at[b, pl.ds(base_r + c * _XR, _XR), :]

    def zero_hist(nwords):
        z = jnp.zeros((_L,), jnp.float32)

        def zb(j, c):
            for u in range(8):
                hist_ref[pl.ds((j * 8 + u) * _L, _L)] = z
            return c

        lax.fori_loop(0, nwords // (_L * 8), zb, 0)

    def stream_pass(shift_hi, path_hi, shift_lo, nb, lane_base):
        def process(in_ref):
            def vb(j, c):
                rcs = []
                for u in range(_UN):
                    jj = j * _UN + u
                    rcs.append((lax.shift_right_logical(jj, 5),
                                (jj & 31) * _L))
                ws = [in_ref[r, pl.ds(c0, _L)] for r, c0 in rcs]
                uis = [plsc.bitcast(w, jnp.int32) for w in ws]
                ms = [lax.shift_right_logical(ui, shift_hi) == path_hi
                      for ui in uis]
                ixs = [lane_base + (lax.shift_right_logical(ui, shift_lo)
                                    & (nb - 1)) for ui in uis]
                for w, m, ix in zip(ws, ms, ixs):
                    plsc.addupdate_scatter(hist_ref, [ix], w, mask=m)
                return c

            lax.fori_loop(0, _CHP // _L // _UN, vb, 0)

        pltpu.async_copy(psrc(0), in_a, sem_a)

        def bodyr(g, carry):
            pltpu.async_copy(psrc(2 * g + 1), in_b, sem_b)
            pltpu.make_async_copy(psrc(2 * g), in_a, sem_a).wait()
            process(in_a)

            @pl.when(g + 1 < _NCH // 2)
            def _():
                pltpu.async_copy(psrc(2 * g + 2), in_a, sem_a)

            pltpu.make_async_copy(psrc(2 * g + 1), in_b, sem_b).wait()
            process(in_b)
            return carry

        lax.fori_loop(0, _NCH // 2, bodyr, 0)

    def process0(in_ref, pw_ref):
        def vb(j, c):
            rcs = []
            for u in range(_UN):
                jj = j * _UN + u
                rcs.append((lax.shift_right_logical(jj, 5),
                            (jj & 31) * _L))
            es = [in_ref[r, 0, pl.ds(c0, _L)] for r, c0 in rcs]
            os_ = [in_ref[r, 1, pl.ds(c0, _L)] for r, c0 in rcs]
            ws = [e * e + o * o for e, o in zip(es, os_)]
            ixs = [lane0 + lax.shift_right_logical(
                plsc.bitcast(w, jnp.int32), 20) for w in ws]
            for (r, c0), w in zip(rcs, ws):
                pw_ref[r, pl.ds(c0, _L)] = w
            for w, ix in zip(ws, ixs):
                plsc.addupdate_scatter(hist_ref, [ix], w)
            return c

        lax.fori_loop(0, (_XR * _W) // _L // _UN, vb, 0)

    zero_hist(_HISTW)
    pltpu.async_copy(xsrc(0), xa, sem_xa)

    def body0(g, carry):
        pltpu.async_copy(xsrc(2 * g + 1), xb, sem_xb)
        pltpu.make_async_copy(xsrc(2 * g), xa, sem_xa).wait()

        @pl.when(g > 0)
        def _():
            pltpu.make_async_copy(pwa, pdst(2 * g - 2), sem_wa).wait()

        process0(xa, pwa)
        pltpu.async_copy(pwa, pdst(2 * g), sem_wa)

        @pl.when(g + 1 < _NCH0 // 2)
        def _():
            pltpu.async_copy(xsrc(2 * g + 2), xa, sem_xa)

        pltpu.make_async_copy(xsrc(2 * g + 1), xb, sem_xb).wait()

        @pl.when(g > 0)
        def _():
            pltpu.make_async_copy(pwb, pdst(2 * g - 1), sem_wb).wait()

        process0(xb, pwb)
        pltpu.async_copy(pwb, pdst(2 * g + 1), sem_wb)
        return carry

    lax.fori_loop(0, _NCH0 // 2, body0, 0)
    pltpu.make_async_copy(pwa, pdst(_NCH0 - 2), sem_wa).wait()
    pltpu.make_async_copy(pwb, pdst(_NCH0 - 1), sem_wb).wait()
    _merge_hist(hist_ref, hc_ref, _NB0, _ST0)
    _merge_partner(hc_ref, pbuf_ref, sh_ref, sid, _NB0)
    total = _total(hc_ref, _NB0)
    target = jnp.float32(_LAM) * (total + jnp.float32(1e-10))
    best0, t0, sa = _scan_level(hc_ref, _NB0, target, jnp.float32(0.0))

    def refine(shift_hi, path_hi, shift_lo, sum_above):
        zero_hist(_L * _NB12 + 128)
        stream_pass(shift_hi, path_hi, shift_lo, _NB12, lane12)
        _merge_hist(hist_ref, hc_ref, _NB12, _ST12)
        _merge_partner(hc_ref, pbuf_ref, sh_ref, sid, _NB12)
        _, t, sa2 = _scan_level(hc_ref, _NB12, target, sum_above)
        return t, sa2

    t1, sa = refine(20, t0, 10, sa)
    path01 = (t0 << 10) | t1
    t2, sa = refine(10, path01, 0, sa)

    vbits = jnp.full((_L,), (path01 << 10) | t2, jnp.int32)
    vf = plsc.bitcast(vbits, jnp.float32)
    include = (jnp.full((_L,), target - sa) >= vf) | jnp.full(
        (_L,), sa <= jnp.float32(0.0))
    thr_bits = vbits + jnp.where(include, jnp.int32(0), jnp.int32(1))
    thrf = plsc.bitcast(thr_bits, jnp.float32)
    no_cross = jnp.full((_L,), best0 < jnp.int32(0))
    thrf = jnp.where(no_cross, jnp.zeros((_L,), jnp.float32), thrf)

    def msrc(c):
        return power_hbm.at[b, pl.ds(base_r + c * _XR, _XR), :]

    def mdst(c):
        return mask_hbm.at[b, pl.ds(base_r + c * _XR, _XR), :]

    mina = in_a.at[pl.ds(0, _XR), :]
    minb = in_b.at[pl.ds(0, _XR), :]
    one = jnp.ones((_L,), jnp.float32)
    zero = jnp.zeros((_L,), jnp.float32)

    def mprocess(in_ref, out_ref2):
        def vb(j, c):
            rcs = []
            for u in range(_UN):
                jj = j * _UN + u
                rcs.append((lax.shift_right_logical(jj, 5),
                            (jj & 31) * _L))
            ws = [in_ref[r, pl.ds(c0, _L)] for r, c0 in rcs]
            vals = [jnp.where(w >= thrf, one, zero) for w in ws]
            for (r, c0), v in zip(rcs, vals):
                out_ref2[r, pl.ds(c0, _L)] = v
            return c

        lax.fori_loop(0, (_XR * _W) // _L // _UN, vb, 0)

    pltpu.async_copy(msrc(0), mina, sem_a)

    def bodym(g, carry):
        pltpu.async_copy(msrc(2 * g + 1), minb, sem_b)
        pltpu.make_async_copy(msrc(2 * g), mina, sem_a).wait()

        @pl.when(g > 0)
        def _():
            pltpu.make_async_copy(pwa, mdst(2 * g - 2), sem_wa).wait()

        mprocess(mina, pwa)
        pltpu.async_copy(pwa, mdst(2 * g), sem_wa)

        @pl.when(g + 1 < _NCH0 // 2)
        def _():
            pltpu.async_copy(msrc(2 * g + 2), mina, sem_a)

        pltpu.make_async_copy(msrc(2 * g + 1), minb, sem_b).wait()

        @pl.when(g > 0)
        def _():
            pltpu.make_async_copy(pwb, mdst(2 * g - 1), sem_wb).wait()

        mprocess(minb, pwb)
        pltpu.async_copy(pwb, mdst(2 * g + 1), sem_wb)
        return carry

    lax.fori_loop(0, _NCH0 // 2, bodym, 0)
    pltpu.make_async_copy(pwa, mdst(_NCH0 - 2), sem_wa).wait()
    pltpu.make_async_copy(pwb, mdst(_NCH0 - 1), sem_wb).wait()


def _sc_select(xp):
    mesh = plsc.VectorSubcoreMesh(core_axis_name="c", subcore_axis_name="s")
    f = functools.partial(
        pl.kernel,
        out_type=(
            jax.ShapeDtypeStruct((_B, _H, _W), jnp.float32),
            jax.ShapeDtypeStruct((_B, _H, _W), jnp.float32),
        ),
        mesh=mesh,
        compiler_params=pltpu.CompilerParams(needs_layout_passes=False),
        scratch_types=[
            pltpu.VMEM((_ROWS, _W), jnp.float32),
            pltpu.VMEM((_ROWS, _W), jnp.float32),
            pltpu.VMEM((_XR, 2, _W), jnp.float32),
            pltpu.VMEM((_XR, 2, _W), jnp.float32),
            pltpu.VMEM((_XR, _W), jnp.float32),
            pltpu.VMEM((_XR, _W), jnp.float32),
            pltpu.VMEM((_HISTW,), jnp.float32),
            pltpu.VMEM((_NB0,), jnp.float32),
            pltpu.VMEM((_NB0,), jnp.float32),
            pltpu.VMEM((_L,), jnp.float32),
            pltpu.VMEM_SHARED((_L, _NB0), jnp.float32),
        ] + [pltpu.SemaphoreType.DMA] * 6,
    )(_sc_body)
    return f(xp)


def kernel(x):
    b, h, w, _ = x.shape
    xp = jnp.transpose(x, (0, 1, 3, 2))
    _, maskf = _sc_select(xp)
    return maskf.reshape(b, h, w, 1)
